# baseline (speedup 1.0000x reference)
import jax
import jax.numpy as jnp
from jax import lax
from jax.experimental import pallas as pl
from jax.experimental.pallas import tpu as pltpu

B = 4096
D = 64
H = B // 2
UNROLL = 8


def _body(idx_s, table_hbm, out_ref, scr_hbm, rows_v, sem_v, sem_h, sem_b):
    def issue(jb, _):
        for u in range(UNROLL):
            j = jb * UNROLL + u
            pltpu.make_async_copy(
                table_hbm.at[pl.ds(idx_s[j], 1), :],
                rows_v.at[pl.ds(j, 1), :], sem_v).start()
            k = H + jb * UNROLL + u
            pltpu.make_async_copy(
                table_hbm.at[pl.ds(idx_s[k], 1), :],
                scr_hbm.at[pl.ds(jb * UNROLL + u, 1), :], sem_h).start()
        return 0

    lax.fori_loop(0, H // UNROLL, issue, 0)
    pltpu.make_async_copy(
        table_hbm.at[pl.ds(0, H), :], scr_hbm, sem_h).wait()
    pltpu.make_async_copy(scr_hbm, rows_v.at[pl.ds(H, H), :], sem_b).start()
    pltpu.make_async_copy(
        table_hbm.at[pl.ds(0, H), :], rows_v.at[pl.ds(0, H), :], sem_v).wait()
    pltpu.make_async_copy(scr_hbm, rows_v.at[pl.ds(H, H), :], sem_b).wait()

    x = rows_v[...]
    rinv = lax.rsqrt(jnp.sum(x * x, axis=1, keepdims=True))
    out_ref[...] = (x * rinv).T


def kernel(nodes, table):
    grid_spec = pltpu.PrefetchScalarGridSpec(
        num_scalar_prefetch=1,
        grid=(1,),
        in_specs=[pl.BlockSpec(memory_space=pl.ANY)],
        out_specs=[
            pl.BlockSpec((D, B), lambda i, idx: (0, 0)),
            pl.BlockSpec(memory_space=pl.ANY),
        ],
        scratch_shapes=[
            pltpu.VMEM((B, D), jnp.float32),
            pltpu.SemaphoreType.DMA,
            pltpu.SemaphoreType.DMA,
            pltpu.SemaphoreType.DMA,
        ],
    )
    out, _ = pl.pallas_call(
        _body,
        grid_spec=grid_spec,
        out_shape=[
            jax.ShapeDtypeStruct((D, B), jnp.float32),
            jax.ShapeDtypeStruct((H, D), jnp.float32),
        ],
    )(nodes.astype(jnp.int32), table)
    return out


# speedup vs baseline: 1.4614x; 1.4614x over previous
import jax
import jax.numpy as jnp
from jax import lax
from jax.experimental import pallas as pl
from jax.experimental.pallas import tpu as pltpu

B = 4096
D = 64
UNROLL = 8


def _body(idx_s, table_hbm, out_ref, rows_v, sem):
    def issue(jb, _):
        for u in range(UNROLL):
            j = jb * UNROLL + u
            pltpu.async_copy(
                table_hbm.at[pl.ds(idx_s[j], 1), :],
                rows_v.at[pl.ds(j, 1), :], sem, priority=u % 2)
        return 0

    lax.fori_loop(0, B // UNROLL, issue, 0)
    pltpu.make_async_copy(table_hbm.at[pl.ds(0, B), :], rows_v, sem).wait()

    x = rows_v[...]
    rinv = lax.rsqrt(jnp.sum(x * x, axis=1, keepdims=True))
    out_ref[...] = (x * rinv).T


def kernel(nodes, table):
    grid_spec = pltpu.PrefetchScalarGridSpec(
        num_scalar_prefetch=1,
        grid=(1,),
        in_specs=[pl.BlockSpec(memory_space=pl.ANY)],
        out_specs=pl.BlockSpec((D, B), lambda i, idx: (0, 0)),
        scratch_shapes=[
            pltpu.VMEM((B, D), jnp.float32),
            pltpu.SemaphoreType.DMA,
        ],
    )
    return pl.pallas_call(
        _body,
        grid_spec=grid_spec,
        out_shape=jax.ShapeDtypeStruct((D, B), jnp.float32),
    )(nodes.astype(jnp.int32), table)
